# Initial kernel scaffold; baseline (speedup 1.0000x reference)
import jax
import jax.numpy as jnp
from jax.experimental import pallas as pl


def kernel(hidden_states, W_router, W_gate, W_up, W_down, Ws_gate_up, Ws_down, W_shared_gate):
    raise NotImplementedError("write your pallas kernel here")


# speedup vs baseline: 1.2411x; 1.2411x over previous
import jax
import jax.numpy as jnp
from jax.experimental import pallas as pl
from jax.experimental.pallas import tpu as pltpu

_T, _D, _E, _K, _FF, _SFF = 2048, 2048, 8, 2, 768, 2048


def _router_body(x_ref, wr_ref, wsg_ref, comb_ref, sgate_ref):
    x = x_ref[...]
    logits = jnp.dot(x, wr_ref[...], preferred_element_type=jnp.float32)
    idx = jax.lax.broadcasted_iota(jnp.int32, logits.shape, 1)
    m1 = jnp.max(logits, axis=1, keepdims=True)
    a1 = jnp.min(jnp.where(logits == m1, idx, _E), axis=1, keepdims=True)
    masked = jnp.where(idx == a1, -jnp.inf, logits)
    m2 = jnp.max(masked, axis=1, keepdims=True)
    a2 = jnp.min(jnp.where(masked == m2, idx, _E), axis=1, keepdims=True)
    w1 = jax.nn.sigmoid(m1 - m2)
    w2 = 1.0 - w1
    comb_ref[...] = jnp.where(idx == a1, w1, 0.0) + jnp.where(idx == a2, w2, 0.0)
    sl = jnp.dot(x, wsg_ref[...], preferred_element_type=jnp.float32)
    sgate_ref[...] = jax.nn.sigmoid(sl)


def _routed_body(comb_ref, xb_ref, wg_ref, wu_ref, wd_ref, acc_ref):
    e = pl.program_id(1)
    xb = xb_ref[...]
    g = jnp.dot(xb, wg_ref[0], preferred_element_type=jnp.float32)
    u = jnp.dot(xb, wu_ref[0], preferred_element_type=jnp.float32)
    h = (g * jax.nn.sigmoid(g)) * u
    comb = comb_ref[...]
    idx = jax.lax.broadcasted_iota(jnp.int32, comb.shape, 1)
    w = jnp.sum(jnp.where(idx == e, comb, 0.0), axis=1, keepdims=True)
    hw = (h * w).astype(jnp.bfloat16)
    contrib = jnp.dot(hw, wd_ref[0], preferred_element_type=jnp.float32)

    @pl.when(e == 0)
    def _():
        acc_ref[...] = contrib

    @pl.when(e != 0)
    def _():
        acc_ref[...] += contrib


def _shared_body(xb_ref, wgu_ref, wd_ref, routed_ref, sgate_ref, out_ref):
    xb = xb_ref[...]
    gu = jnp.dot(xb, wgu_ref[...], preferred_element_type=jnp.float32)
    sg = gu[:, :_SFF]
    su = gu[:, _SFF:]
    hs = ((sg * jax.nn.sigmoid(sg)) * su).astype(jnp.bfloat16)
    sh = jnp.dot(hs, wd_ref[...], preferred_element_type=jnp.float32)
    out_ref[...] = routed_ref[...] + sgate_ref[...] * sh


def kernel(hidden_states, W_router, W_gate, W_up, W_down, Ws_gate_up, Ws_down,
           W_shared_gate):
    x = hidden_states
    xb = x.astype(jnp.bfloat16)
    wg = W_gate.astype(jnp.bfloat16)
    wu = W_up.astype(jnp.bfloat16)
    wd = W_down.astype(jnp.bfloat16)
    wsgu = Ws_gate_up.astype(jnp.bfloat16)
    wsd = Ws_down.astype(jnp.bfloat16)

    comb, sgate = pl.pallas_call(
        _router_body,
        grid=(1,),
        in_specs=[
            pl.BlockSpec((_T, _D), lambda i: (0, 0)),
            pl.BlockSpec((_D, _E), lambda i: (0, 0)),
            pl.BlockSpec((_D, 1), lambda i: (0, 0)),
        ],
        out_specs=[
            pl.BlockSpec((_T, _E), lambda i: (0, 0)),
            pl.BlockSpec((_T, 1), lambda i: (0, 0)),
        ],
        out_shape=[
            jax.ShapeDtypeStruct((_T, _E), jnp.float32),
            jax.ShapeDtypeStruct((_T, 1), jnp.float32),
        ],
    )(x, W_router, W_shared_gate)

    bt = 1024
    routed = pl.pallas_call(
        _routed_body,
        grid=(_T // bt, _E),
        in_specs=[
            pl.BlockSpec((bt, _E), lambda t, e: (t, 0)),
            pl.BlockSpec((bt, _D), lambda t, e: (t, 0)),
            pl.BlockSpec((1, _D, _FF), lambda t, e: (e, 0, 0)),
            pl.BlockSpec((1, _D, _FF), lambda t, e: (e, 0, 0)),
            pl.BlockSpec((1, _FF, _D), lambda t, e: (e, 0, 0)),
        ],
        out_specs=pl.BlockSpec((bt, _D), lambda t, e: (t, 0)),
        out_shape=jax.ShapeDtypeStruct((_T, _D), jnp.float32),
        compiler_params=pltpu.CompilerParams(
            dimension_semantics=("arbitrary", "arbitrary")),
    )(comb, xb, wg, wu, wd)

    bs = 512
    out = pl.pallas_call(
        _shared_body,
        grid=(_T // bs,),
        in_specs=[
            pl.BlockSpec((bs, _D), lambda t: (t, 0)),
            pl.BlockSpec((_D, 2 * _SFF), lambda t: (0, 0)),
            pl.BlockSpec((_SFF, _D), lambda t: (0, 0)),
            pl.BlockSpec((bs, _D), lambda t: (t, 0)),
            pl.BlockSpec((bs, 1), lambda t: (t, 0)),
        ],
        out_specs=pl.BlockSpec((bs, _D), lambda t: (t, 0)),
        out_shape=jax.ShapeDtypeStruct((_T, _D), jnp.float32),
    )(xb, wsgu, wsd, routed, sgate)
    return out
